# baseline (speedup 1.0000x reference)
import functools
import math

import jax
import jax.numpy as jnp
from jax import lax
from jax.experimental import pallas as pl
from jax.experimental.pallas import tpu as pltpu
from jax.experimental.pallas import tpu_sc as plsc

B = 16384
D = 128
L = 16
NC, NS = 2, 16
NW = NC * NS
ROWS_PER_W = B // NW
R = 128
NCHUNK = ROWS_PER_W // R
NBUF = 3
NCLS = 1000
TROWS = 64
HALF_LOG_2PI = 0.5 * math.log(2.0 * math.pi)

_mesh = plsc.VectorSubcoreMesh(core_axis_name="c", subcore_axis_name="s")

TPAD = 1024

_scratch = [
    pltpu.VMEM_SHARED((TPAD, D), jnp.float32),
    pltpu.VMEM((NBUF, R), jnp.int32),
    pltpu.VMEM((NBUF, R, D), jnp.float32),
    pltpu.VMEM((NBUF, R, D), jnp.float32),
    pltpu.VMEM((NBUF, R), jnp.float32),
    pltpu.VMEM((R * L,), jnp.float32),
] + [pltpu.SemaphoreType.DMA] * (4 * NBUF + 1)


@functools.partial(
    pl.kernel,
    mesh=_mesh,
    compiler_params=pltpu.CompilerParams(needs_layout_passes=False),
    out_type=[
        jax.ShapeDtypeStruct((B,), jnp.float32),
        jax.ShapeDtypeStruct((B, D), jnp.float32),
    ],
    scratch_types=_scratch,
)
def _sc_logprob(x_hbm, y_hbm, muf_hbm, loss_hbm, lu_hbm,
                stab, idx_v, x_v, o_v, l_v, acc_v, *sems):
    sem_i = sems[0:NBUF]
    sem_x = sems[NBUF:2 * NBUF]
    sem_g = sems[2 * NBUF:3 * NBUF]
    sem_o = sems[3 * NBUF:4 * NBUF]
    sem_p = sems[4 * NBUF]
    sid = lax.axis_index("s")
    wid = sid * NC + lax.axis_index("c")
    base = wid * ROWS_PER_W
    col_idx = lax.iota(jnp.int32, L) * L

    def issue_in(ci):
        b = ci % NBUF
        off = base + ci * R
        pltpu.async_copy(y_hbm.at[pl.ds(off, R)], idx_v.at[b], sem_i[b])
        pltpu.async_copy(x_hbm.at[pl.ds(off, R), :], x_v.at[b], sem_x[b])

    def issue_gather(ci):
        b = ci % NBUF
        off = base + ci * R
        pltpu.make_async_copy(y_hbm.at[pl.ds(off, R)], idx_v.at[b],
                              sem_i[b]).wait()
        pltpu.make_async_copy(x_hbm.at[pl.ds(off, R), :], x_v.at[b],
                              sem_x[b]).wait()
        pltpu.async_copy(stab.at[idx_v.at[b]], x_v.at[b], sem_g[b],
                         add=True)

    def compute(ci):
        b = ci % NBUF
        pltpu.make_async_copy(stab.at[idx_v.at[b]], x_v.at[b],
                              sem_g[b]).wait()

        @plsc.parallel_loop(0, R // L)
        def group_body(gi):
            gb = gi * (L * L)
            for r16 in range(L):
                row = gi * L + r16
                acc0 = jnp.zeros((L,), jnp.float32)
                acc1 = jnp.zeros((L,), jnp.float32)
                for j in range(D // L):
                    d = x_v[b, row, pl.ds(j * L, L)]
                    o = 0.5 * (d * d) + HALF_LOG_2PI
                    if j % 2 == 0:
                        acc0 = acc0 + o
                    else:
                        acc1 = acc1 + o
                    o_v[b, row, pl.ds(j * L, L)] = o
                acc_v[pl.ds(gb + r16 * L, L)] = acc0 + acc1
            rowsums = jnp.zeros((L,), jnp.float32)
            for c in range(L):
                rowsums = rowsums + plsc.load_gather(acc_v, [gb + col_idx + c])
            l_v[b, pl.ds(gi * L, L)] = rowsums

    def issue_out(ci):
        b = ci % NBUF
        off = base + ci * R
        pltpu.async_copy(o_v.at[b], lu_hbm.at[pl.ds(off, R), :], sem_o[b])
        pltpu.async_copy(l_v.at[b], loss_hbm.at[pl.ds(off, R)], sem_o[b])

    def wait_out(ci):
        b = ci % NBUF
        off = base + ci * R
        pltpu.make_async_copy(o_v.at[b], lu_hbm.at[pl.ds(off, R), :],
                              sem_o[b]).wait()
        pltpu.make_async_copy(l_v.at[b], loss_hbm.at[pl.ds(off, R)],
                              sem_o[b]).wait()

    toff = sid * TROWS
    pltpu.async_copy(muf_hbm.at[pl.ds(toff, TROWS), :],
                     stab.at[pl.ds(toff, TROWS), :], sem_p)
    issue_in(0)
    if NCHUNK > 1:
        issue_in(1)
    pltpu.make_async_copy(muf_hbm.at[pl.ds(toff, TROWS), :],
                          stab.at[pl.ds(toff, TROWS), :], sem_p).wait()
    plsc.subcore_barrier()
    issue_gather(0)

    for ci in range(NCHUNK):
        if ci + 2 < NCHUNK:
            issue_in(ci + 2)
        if ci + 1 < NCHUNK:
            issue_gather(ci + 1)
        if ci >= NBUF:
            wait_out(ci - NBUF)
        compute(ci)
        issue_out(ci)
    for ci in range(max(0, NCHUNK - NBUF), NCHUNK):
        wait_out(ci)


def kernel(x, y, mu0, mu):
    ntab_pad = -(jnp.pad(mu, ((1, TPAD - NCLS), (0, 0))) + mu0[None, :])
    loss, loss_unsummed = _sc_logprob(x, y.astype(jnp.int32), ntab_pad)
    return (loss, loss_unsummed)


# speedup vs baseline: 1.0317x; 1.0317x over previous
import functools
import math

import jax
import jax.numpy as jnp
from jax import lax
from jax.experimental import pallas as pl
from jax.experimental.pallas import tpu as pltpu
from jax.experimental.pallas import tpu_sc as plsc

B = 16384
D = 128
L = 16
NC, NS = 2, 16
NW = NC * NS
ROWS_PER_W = B // NW
R = 128
NCHUNK = ROWS_PER_W // R
NBUF = 3
NCLS = 1000
TROWS = 64
HALF_LOG_2PI = 0.5 * math.log(2.0 * math.pi)

_mesh = plsc.VectorSubcoreMesh(core_axis_name="c", subcore_axis_name="s")

TPAD = 1024

_scratch = [
    pltpu.VMEM_SHARED((TPAD, D), jnp.float32),
    pltpu.VMEM((NBUF, R), jnp.int32),
    pltpu.VMEM((NBUF, R, D), jnp.float32),
    pltpu.VMEM((NBUF, R, D), jnp.float32),
    pltpu.VMEM((NBUF, R), jnp.float32),
    pltpu.VMEM((R * L,), jnp.float32),
] + [pltpu.SemaphoreType.DMA] * (4 * NBUF + 1)


@functools.partial(
    pl.kernel,
    mesh=_mesh,
    compiler_params=pltpu.CompilerParams(needs_layout_passes=False),
    out_type=[
        jax.ShapeDtypeStruct((B,), jnp.float32),
        jax.ShapeDtypeStruct((B, D), jnp.float32),
    ],
    scratch_types=_scratch,
)
def _sc_logprob(x_hbm, y_hbm, muf_hbm, loss_hbm, lu_hbm,
                stab, idx_v, x_v, o_v, l_v, acc_v, *sems):
    sem_i = sems[0:NBUF]
    sem_x = sems[NBUF:2 * NBUF]
    sem_g = sems[2 * NBUF:3 * NBUF]
    sem_o = sems[3 * NBUF:4 * NBUF]
    sem_p = sems[4 * NBUF]
    sid = lax.axis_index("s")
    wid = sid * NC + lax.axis_index("c")
    base = wid * ROWS_PER_W
    col_idx = lax.iota(jnp.int32, L) * L

    def issue_in(ci):
        b = ci % NBUF
        off = base + ci * R
        pltpu.async_copy(y_hbm.at[pl.ds(off, R)], idx_v.at[b], sem_i[b])
        pltpu.async_copy(x_hbm.at[pl.ds(off, R), :], x_v.at[b], sem_x[b])

    def issue_gather(ci):
        b = ci % NBUF
        off = base + ci * R
        pltpu.make_async_copy(y_hbm.at[pl.ds(off, R)], idx_v.at[b],
                              sem_i[b]).wait()
        pltpu.make_async_copy(x_hbm.at[pl.ds(off, R), :], x_v.at[b],
                              sem_x[b]).wait()
        pltpu.async_copy(stab.at[idx_v.at[b]], x_v.at[b], sem_g[b],
                         add=True)

    def compute(ci):
        b = ci % NBUF
        pltpu.make_async_copy(stab.at[idx_v.at[b]], x_v.at[b],
                              sem_g[b]).wait()

        def group_body(gi, carry):
            gb = gi * (L * L)
            for r16 in range(L):
                row = gi * L + r16
                acc0 = jnp.zeros((L,), jnp.float32)
                acc1 = jnp.zeros((L,), jnp.float32)
                for j in range(D // L):
                    d = x_v[b, row, pl.ds(j * L, L)]
                    o = 0.5 * (d * d) + HALF_LOG_2PI
                    if j % 2 == 0:
                        acc0 = acc0 + o
                    else:
                        acc1 = acc1 + o
                    o_v[b, row, pl.ds(j * L, L)] = o
                acc_v[pl.ds(gb + r16 * L, L)] = acc0 + acc1
            rowsums = jnp.zeros((L,), jnp.float32)
            for c in range(L):
                rowsums = rowsums + plsc.load_gather(acc_v, [gb + col_idx + c])
            l_v[b, pl.ds(gi * L, L)] = rowsums
            return carry

        lax.fori_loop(0, R // L, group_body, 0)

    def issue_out(ci):
        b = ci % NBUF
        off = base + ci * R
        pltpu.async_copy(o_v.at[b], lu_hbm.at[pl.ds(off, R), :], sem_o[b])
        pltpu.async_copy(l_v.at[b], loss_hbm.at[pl.ds(off, R)], sem_o[b])

    def wait_out(ci):
        b = ci % NBUF
        off = base + ci * R
        pltpu.make_async_copy(o_v.at[b], lu_hbm.at[pl.ds(off, R), :],
                              sem_o[b]).wait()
        pltpu.make_async_copy(l_v.at[b], loss_hbm.at[pl.ds(off, R)],
                              sem_o[b]).wait()

    toff = sid * TROWS
    pltpu.async_copy(muf_hbm.at[pl.ds(toff, TROWS), :],
                     stab.at[pl.ds(toff, TROWS), :], sem_p)
    issue_in(0)
    if NCHUNK > 1:
        issue_in(1)
    pltpu.make_async_copy(muf_hbm.at[pl.ds(toff, TROWS), :],
                          stab.at[pl.ds(toff, TROWS), :], sem_p).wait()
    plsc.subcore_barrier()
    issue_gather(0)

    for ci in range(NCHUNK):
        if ci + 2 < NCHUNK:
            issue_in(ci + 2)
        if ci + 1 < NCHUNK:
            issue_gather(ci + 1)
        if ci >= NBUF:
            wait_out(ci - NBUF)
        compute(ci)
        issue_out(ci)
    for ci in range(max(0, NCHUNK - NBUF), NCHUNK):
        wait_out(ci)


def kernel(x, y, mu0, mu):
    ntab_pad = -(jnp.pad(mu, ((1, TPAD - NCLS), (0, 0))) + mu0[None, :])
    loss, loss_unsummed = _sc_logprob(x, y.astype(jnp.int32), ntab_pad)
    return (loss, loss_unsummed)
